# baseline (speedup 1.0000x reference)
import functools

import jax
import jax.numpy as jnp
from jax import lax
from jax.experimental import pallas as pl
from jax.experimental.pallas import tpu as pltpu
from jax.experimental.pallas import tpu_sc as plsc

NC, NS, L = 2, 16, 16
NW = NC * NS
NPAIRS = 160000
PAD = 163840
PER_W = PAD // NW
C = 160
NCHUNK = PER_W // C
D = 256
DW = D // 2
V = 10000
MARGIN = 0.5


def _norm_body(h_ref, o_ref, hn_ref, on_ref):
    for src, dst in ((h_ref, hn_ref), (o_ref, on_ref)):
        x = src[...]
        n = jnp.sqrt(jnp.sum(x * x, axis=-1, keepdims=True))
        dst[...] = (x / jnp.maximum(n, 1e-12)).astype(jnp.bfloat16)


_normalize_tables = pl.pallas_call(
    _norm_body,
    grid=(10,),
    in_specs=[pl.BlockSpec((V // 10, D), lambda i: (i, 0))] * 2,
    out_specs=[pl.BlockSpec((V // 10, D), lambda i: (i, 0))] * 2,
    out_shape=[jax.ShapeDtypeStruct((V, D), jnp.bfloat16)] * 2,
)

_mesh = plsc.VectorSubcoreMesh(
    core_axis_name="c", subcore_axis_name="s", num_cores=NC, num_subcores=NS
)


@functools.partial(
    pl.kernel,
    out_type=jax.ShapeDtypeStruct((NW, L), jnp.float32),
    mesh=_mesh,
    compiler_params=pltpu.CompilerParams(
        use_tc_tiling_on_sc=False, needs_layout_passes=False),
    scratch_types=[
        pltpu.VMEM((PER_W,), jnp.int32),
        pltpu.VMEM((PER_W,), jnp.int32),
        pltpu.VMEM((PER_W + L,), jnp.float32),
        pltpu.VMEM((C, DW), jnp.int32),
        pltpu.VMEM((C, DW), jnp.int32),
        pltpu.VMEM((C, DW), jnp.int32),
        pltpu.VMEM((C, DW), jnp.int32),
        pltpu.VMEM((L,), jnp.float32),
        pltpu.SemaphoreType.DMA,
        pltpu.SemaphoreType.DMA,
        pltpu.SemaphoreType.DMA,
        pltpu.SemaphoreType.DMA,
    ],
)
def _sc_loss(hn_hbm, on_hbm, ia_hbm, ib_hbm, cf_hbm, out_hbm,
             ia_v, ib_v, cf_v, h_v0, h_v1, o_v0, o_v1, stage_v,
             sem_a0, sem_a1, sem_b0, sem_b1):
    wid = lax.axis_index("s") * NC + lax.axis_index("c")
    base_w = wid * PER_W

    pltpu.sync_copy(ia_hbm.at[pl.ds(base_w, PER_W)], ia_v)
    pltpu.sync_copy(ib_hbm.at[pl.ds(base_w, PER_W)], ib_v)
    pltpu.sync_copy(cf_hbm.at[pl.ds(base_w, PER_W)], cf_v.at[pl.ds(0, PER_W)])

    bufs = ((h_v0, o_v0, sem_a0, sem_b0), (h_v1, o_v1, sem_a1, sem_b1))

    def start_pair(off, b):
        hb, ob, sa, sb = bufs[b]
        pltpu.async_copy(hn_hbm.at[ia_v.at[pl.ds(off, C)]], hb, sa)
        pltpu.async_copy(on_hbm.at[ib_v.at[pl.ds(off, C)]], ob, sb)

    def wait_pair(b):
        hb, ob, sa, sb = bufs[b]
        pltpu.make_async_copy(hn_hbm.at[ia_v.at[pl.ds(0, C)]], hb, sa).wait()
        pltpu.make_async_copy(on_hbm.at[ib_v.at[pl.ds(0, C)]], ob, sb).wait()

    def compute_chunk(off, b, acc):
        hb, ob, _, _ = bufs[b]

        def pair_body(p, a):
            prods = []
            for j in range(DW // L):
                hw = plsc.bitcast(hb[p, pl.ds(j * L, L)], jnp.bfloat16)
                ow = plsc.bitcast(ob[p, pl.ds(j * L, L)], jnp.bfloat16)
                pa, pb = plsc.unpack(hw * ow,
                                     format=plsc.PackFormat.INTERLEAVED)
                prods.append(pa + pb)
            while len(prods) > 1:
                prods = [prods[k] + prods[k + 1]
                         for k in range(0, len(prods) - 1, 2)] + (
                             [prods[-1]] if len(prods) % 2 else [])
            s = jnp.sum(prods[0])
            cv = cf_v[pl.ds(off + p, L)]
            lv = jnp.maximum(MARGIN - s, 0.0) * cv[0]
            return a + jnp.full((L,), lv, jnp.float32)

        return lax.fori_loop(0, C, pair_body, acc)

    start_pair(0, 0)

    def outer_body(i2, acc):
        for b in range(2):
            chunk = 2 * i2 + b
            wait_pair(b)

            @pl.when(chunk + 1 < NCHUNK)
            def _():
                start_pair((chunk + 1) * C, 1 - b)

            acc = compute_chunk(chunk * C, b, acc)
        return acc

    loss = lax.fori_loop(0, NCHUNK // 2, outer_body,
                         jnp.zeros((L,), jnp.float32))

    stage_v[...] = loss
    pltpu.sync_copy(stage_v, out_hbm.at[wid])


def _fin_body(p_ref, c_ref):
    c_ref[...] = jnp.full(
        (1, 1), jnp.sum(p_ref[...]) * (1.0 / (NPAIRS * L)), jnp.float32)


_finalize = pl.pallas_call(
    _fin_body,
    out_shape=jax.ShapeDtypeStruct((1, 1), jnp.float32),
)


def kernel(human_gene_embeddings, ortholog_embeddings, ortholog_pairs,
           confidence_scores):
    hn, on = _normalize_tables(human_gene_embeddings, ortholog_embeddings)
    hn = lax.bitcast_convert_type(hn.reshape(V, DW, 2), jnp.int32)
    on = lax.bitcast_convert_type(on.reshape(V, DW, 2), jnp.int32)
    npad = PAD - NPAIRS
    ia = jnp.concatenate(
        [ortholog_pairs[:, 0], jnp.zeros((npad,), jnp.int32)])
    ib = jnp.concatenate(
        [ortholog_pairs[:, 1], jnp.zeros((npad,), jnp.int32)])
    cf = jnp.concatenate(
        [confidence_scores, jnp.zeros((npad,), jnp.float32)])
    parts = _sc_loss(hn, on, ia, ib, cf)
    return _finalize(parts)[0, 0]


# speedup vs baseline: 1.2851x; 1.2851x over previous
import functools

import jax
import jax.numpy as jnp
from jax import lax
from jax.experimental import pallas as pl
from jax.experimental.pallas import tpu as pltpu
from jax.experimental.pallas import tpu_sc as plsc

NC, NS, L = 2, 16, 16
NW = NC * NS
NPAIRS = 160000
PAD = 163840
PER_W = PAD // NW
C = 80
NCHUNK = PER_W // C
D = 256
V = 10000
MARGIN = 0.5


def _norm_body(h_ref, o_ref, hn_ref, on_ref):
    for src, dst in ((h_ref, hn_ref), (o_ref, on_ref)):
        x = src[...]
        n = jnp.sqrt(jnp.sum(x * x, axis=-1, keepdims=True))
        dst[...] = x / jnp.maximum(n, 1e-12)


_normalize_tables = pl.pallas_call(
    _norm_body,
    grid=(10,),
    in_specs=[pl.BlockSpec((V // 10, D), lambda i: (i, 0))] * 2,
    out_specs=[pl.BlockSpec((V // 10, D), lambda i: (i, 0))] * 2,
    out_shape=[jax.ShapeDtypeStruct((V, D), jnp.float32)] * 2,
)

_mesh = plsc.VectorSubcoreMesh(
    core_axis_name="c", subcore_axis_name="s", num_cores=NC, num_subcores=NS
)


@functools.partial(
    pl.kernel,
    out_type=jax.ShapeDtypeStruct((NW, L), jnp.float32),
    mesh=_mesh,
    compiler_params=pltpu.CompilerParams(
        use_tc_tiling_on_sc=False, needs_layout_passes=False),
    scratch_types=[
        pltpu.VMEM((PER_W,), jnp.int32),
        pltpu.VMEM((PER_W,), jnp.int32),
        pltpu.VMEM((PER_W + L,), jnp.float32),
        pltpu.VMEM((C, D), jnp.float32),
        pltpu.VMEM((C, D), jnp.float32),
        pltpu.VMEM((C, D), jnp.float32),
        pltpu.VMEM((C, D), jnp.float32),
        pltpu.VMEM((L,), jnp.float32),
        pltpu.SemaphoreType.DMA,
        pltpu.SemaphoreType.DMA,
        pltpu.SemaphoreType.DMA,
        pltpu.SemaphoreType.DMA,
    ],
)
def _sc_loss(hn_hbm, on_hbm, ia_hbm, ib_hbm, cf_hbm, out_hbm,
             ia_v, ib_v, cf_v, h_v0, h_v1, o_v0, o_v1, stage_v,
             sem_a0, sem_a1, sem_b0, sem_b1):
    wid = lax.axis_index("s") * NC + lax.axis_index("c")
    base_w = wid * PER_W

    pltpu.sync_copy(ia_hbm.at[pl.ds(base_w, PER_W)], ia_v)
    pltpu.sync_copy(ib_hbm.at[pl.ds(base_w, PER_W)], ib_v)
    pltpu.sync_copy(cf_hbm.at[pl.ds(base_w, PER_W)], cf_v.at[pl.ds(0, PER_W)])

    bufs = ((h_v0, o_v0, sem_a0, sem_b0), (h_v1, o_v1, sem_a1, sem_b1))

    def start_pair(off, b):
        hb, ob, sa, sb = bufs[b]
        pltpu.async_copy(hn_hbm.at[ia_v.at[pl.ds(off, C)]], hb, sa)
        pltpu.async_copy(on_hbm.at[ib_v.at[pl.ds(off, C)]], ob, sb)

    def wait_pair(b):
        hb, ob, sa, sb = bufs[b]
        pltpu.make_async_copy(hn_hbm.at[ia_v.at[pl.ds(0, C)]], hb, sa).wait()
        pltpu.make_async_copy(on_hbm.at[ib_v.at[pl.ds(0, C)]], ob, sb).wait()

    def compute_chunk(off, b, acc):
        hb, ob, _, _ = bufs[b]

        def pair_body(p, a):
            prods = [hb[p, pl.ds(j * L, L)] * ob[p, pl.ds(j * L, L)]
                     for j in range(D // L)]
            while len(prods) > 1:
                prods = [prods[k] + prods[k + 1]
                         for k in range(0, len(prods) - 1, 2)] + (
                             [prods[-1]] if len(prods) % 2 else [])
            s = jnp.sum(prods[0])
            cv = cf_v[pl.ds(off + p, L)]
            lv = jnp.maximum(MARGIN - s, 0.0) * cv[0]
            return a + jnp.full((L,), lv, jnp.float32)

        return lax.fori_loop(0, C, pair_body, acc)

    start_pair(0, 0)

    def outer_body(i2, acc):
        for b in range(2):
            chunk = 2 * i2 + b
            wait_pair(b)

            @pl.when(chunk + 1 < NCHUNK)
            def _():
                start_pair((chunk + 1) * C, 1 - b)

            acc = compute_chunk(chunk * C, b, acc)
        return acc

    loss = lax.fori_loop(0, NCHUNK // 2, outer_body,
                         jnp.zeros((L,), jnp.float32))

    stage_v[...] = loss
    pltpu.sync_copy(stage_v, out_hbm.at[wid])


def _fin_body(p_ref, c_ref):
    c_ref[...] = jnp.full(
        (1, 1), jnp.sum(p_ref[...]) * (1.0 / (NPAIRS * L)), jnp.float32)


_finalize = pl.pallas_call(
    _fin_body,
    out_shape=jax.ShapeDtypeStruct((1, 1), jnp.float32),
)


def kernel(human_gene_embeddings, ortholog_embeddings, ortholog_pairs,
           confidence_scores):
    hn, on = _normalize_tables(human_gene_embeddings, ortholog_embeddings)
    npad = PAD - NPAIRS
    ia = jnp.concatenate(
        [ortholog_pairs[:, 0], jnp.zeros((npad,), jnp.int32)])
    ib = jnp.concatenate(
        [ortholog_pairs[:, 1], jnp.zeros((npad,), jnp.int32)])
    cf = jnp.concatenate(
        [confidence_scores, jnp.zeros((npad,), jnp.float32)])
    parts = _sc_loss(hn, on, ia, ib, cf)
    return _finalize(parts)[0, 0]
